# baseline (speedup 1.0000x reference)
import functools

import jax
import jax.numpy as jnp
from jax import lax
from jax.experimental import pallas as pl
from jax.experimental.pallas import tpu as pltpu
from jax.experimental.pallas import tpu_sc as plsc

N = 4096
D = 512
E = 131072


def _mm_body(a_ref, b_ref, scale_ref, bias_ref, *refs, trans_lhs,
             scale_rows, bias, leaky, bm, bf16, out_bf16):
    if out_bf16:
        out_ref, out2_ref, acc_ref = refs
    else:
        out_ref, acc_ref = refs
    k = pl.program_id(2)

    @pl.when(k == 0)
    def _():
        acc_ref[...] = jnp.zeros_like(acc_ref)

    a, b = a_ref[...], b_ref[...]
    if bf16:
        a, b = a.astype(jnp.bfloat16), b.astype(jnp.bfloat16)
    if trans_lhs:
        acc_ref[...] += jax.lax.dot_general(
            a, b, (((0,), (0,)), ((), ())),
            preferred_element_type=jnp.float32)
    else:
        acc_ref[...] += jnp.dot(a, b, preferred_element_type=jnp.float32)

    @pl.when(k == pl.num_programs(2) - 1)
    def _():
        acc = acc_ref[...]
        if bias:
            j = pl.program_id(1)
            bn = out_ref.shape[1]
            acc = acc + bias_ref[pl.ds(j * bn, bn)][None, :]
        if scale_rows:
            i = pl.program_id(0)
            acc = acc * scale_ref[pl.ds(i * bm, bm)][:, None]
        if leaky:
            acc = jnp.where(acc > 0, acc, 0.01 * acc)
        out_ref[...] = acc
        if out_bf16:
            out2_ref[...] = acc.astype(jnp.bfloat16)


def _mm(a, b, *, scale=None, bias=None, leaky=False, trans_lhs=False,
        bf16=False, out_bf16=False, bm=512, bn=512, bk=512):
    if trans_lhs:
        ka, m = a.shape
    else:
        m, ka = a.shape
    kb, n = b.shape
    assert ka == kb
    grid = (m // bm, n // bn, ka // bk)
    in_specs = [
        pl.BlockSpec((bk, bm) if trans_lhs else (bm, bk),
                     (lambda i, j, k: (k, i)) if trans_lhs
                     else (lambda i, j, k: (i, k))),
        pl.BlockSpec((bk, bn), lambda i, j, k: (k, j)),
        pl.BlockSpec((m,), lambda i, j, k: (0,)),
        pl.BlockSpec((n,), lambda i, j, k: (0,)),
    ]
    scale_arr = scale if scale is not None else jnp.zeros((m,), jnp.float32)
    bias_arr = bias if bias is not None else jnp.zeros((n,), jnp.float32)
    body = functools.partial(_mm_body, trans_lhs=trans_lhs,
                             scale_rows=scale is not None,
                             bias=bias is not None, leaky=leaky, bm=bm,
                             bf16=bf16, out_bf16=out_bf16)
    out_specs = [pl.BlockSpec((bm, bn), lambda i, j, k: (i, j))]
    out_shape = [jax.ShapeDtypeStruct((m, n), jnp.float32)]
    if out_bf16:
        out_specs.append(pl.BlockSpec((bm, bn), lambda i, j, k: (i, j)))
        out_shape.append(jax.ShapeDtypeStruct((m, n), jnp.bfloat16))
    res = pl.pallas_call(
        body,
        grid=grid,
        in_specs=in_specs,
        out_specs=out_specs,
        out_shape=out_shape,
        scratch_shapes=[pltpu.VMEM((bm, bn), jnp.float32)],
        compiler_params=pltpu.CompilerParams(
            dimension_semantics=("parallel", "parallel", "arbitrary")),
    )(a, b, scale_arr, bias_arr)
    return res if out_bf16 else res[0]


def _agg_body(*refs, norm, leaky, bf16):
    if norm:
        a_ref, t_ref, td_ref, dis_ref, bias_ref, out_ref, acc_ref = refs
    else:
        a_ref, t_ref, out_ref, acc_ref = refs
    k = pl.program_id(0)

    @pl.when(k == 0)
    def _():
        acc_ref[...] = jnp.zeros_like(acc_ref)

    a, t = a_ref[...], t_ref[...]
    if bf16:
        a, t = a.astype(jnp.bfloat16), t.astype(jnp.bfloat16)
    acc_ref[...] += jnp.dot(a, t, preferred_element_type=jnp.float32)

    @pl.when(k == pl.num_programs(0) - 1)
    def _():
        acc = acc_ref[...]
        if norm:
            acc = (acc + td_ref[...]) * dis_ref[...][:, None]
            acc = acc + bias_ref[...][None, :]
        if leaky:
            acc = jnp.where(acc > 0, acc, 0.01 * acc)
        out_ref[...] = acc


def _agg(a0, ts, dis=None, bias=None, *, leaky=False, bf16=False, bk=512):
    n, d = ts.shape
    norm = dis is not None
    body = functools.partial(_agg_body, norm=norm, leaky=leaky, bf16=bf16)
    in_specs = [
        pl.BlockSpec((n, bk), lambda k: (0, k)),
        pl.BlockSpec((bk, d), lambda k: (k, 0)),
    ]
    args = [a0, ts]
    if norm:
        in_specs += [
            pl.BlockSpec((n, d), lambda k: (0, 0)),
            pl.BlockSpec((n,), lambda k: (0,)),
            pl.BlockSpec((d,), lambda k: (0,)),
        ]
        args += [ts, dis, bias]
    return pl.pallas_call(
        body,
        grid=(n // bk,),
        in_specs=in_specs,
        out_specs=pl.BlockSpec((n, d), lambda k: (0, 0)),
        out_shape=jax.ShapeDtypeStruct((n, d), jnp.float32),
        scratch_shapes=[pltpu.VMEM((n, d), jnp.float32)],
        compiler_params=pltpu.CompilerParams(
            dimension_semantics=("arbitrary",)),
    )(*args)


def _rowscale_body(t_ref, dis_ref, out_ref, *, bm):
    i = pl.program_id(0)
    out_ref[...] = t_ref[...] * dis_ref[pl.ds(i * bm, bm)][:, None]


def _rowscale(t, dis, *, bm=512):
    n, d = t.shape
    return pl.pallas_call(
        functools.partial(_rowscale_body, bm=bm),
        grid=(n // bm,),
        in_specs=[pl.BlockSpec((bm, d), lambda i: (i, 0)),
                  pl.BlockSpec((n,), lambda i: (0,))],
        out_specs=pl.BlockSpec((bm, d), lambda i: (i, 0)),
        out_shape=jax.ShapeDtypeStruct((n, d), jnp.float32),
    )(t, dis)


def _prep_body(a_ref, bf_ref, dis_ref):
    a = a_ref[...]
    bf_ref[...] = a.astype(jnp.bfloat16)
    dis_ref[...] = jax.lax.rsqrt(jnp.sum(a, axis=1) + 1.0)


def _prep(a0, *, bm=512):
    n = a0.shape[0]
    return pl.pallas_call(
        _prep_body,
        grid=(n // bm,),
        in_specs=[pl.BlockSpec((bm, n), lambda i: (i, 0))],
        out_specs=[pl.BlockSpec((bm, n), lambda i: (i, 0)),
                   pl.BlockSpec((bm,), lambda i: (i,))],
        out_shape=[jax.ShapeDtypeStruct((n, n), jnp.bfloat16),
                   jax.ShapeDtypeStruct((n,), jnp.float32)],
    )(a0)


def _conv_pass(t1, a0, params):
    (_, b1, W2, b2, W3, b3, W4, b4, W5, b5, W6, b6) = params
    a0bf, dis = _prep(a0)
    f = _agg(a0bf, _rowscale(t1, dis), dis, b1, leaky=False, bf16=True)
    for (W, b) in ((W2, b2), (W3, b3), (W4, b4), (W5, b5)):
        ts = _mm(f, W, scale=dis, bf16=True)
        f = _agg(a0bf, ts, dis, b, leaky=True, bf16=True)
    g = _agg(a0bf, f, bf16=True)
    return _mm(g, W6, bias=b6, leaky=True, bf16=True, bm=1024, bn=1024,
               out_bf16=True)


_ROWS = 256
_PASSES = N // (_ROWS * 2)
_EPT = E // 16
_WPT = _ROWS * N // 16
_DUMMY = 8192


def _build_a0(edge_index):
    mesh = plsc.VectorSubcoreMesh(core_axis_name="c", subcore_axis_name="s")

    @functools.partial(
        pl.kernel,
        out_type=jax.ShapeDtypeStruct((N * N,), jnp.float32),
        mesh=mesh,
        scratch_types=[
            pltpu.VMEM_SHARED((_ROWS * N + _DUMMY,), jnp.float32),
            pltpu.VMEM((_EPT,), jnp.int32),
            pltpu.VMEM((_EPT,), jnp.int32),
            pltpu.VMEM((_EPT // 128, 128), jnp.int32),
            pltpu.VMEM((128,), jnp.float32),
        ],
    )
    def k(ei_hbm, zeros_hbm, ones_hbm, a0_hbm, acc, srcv, dstv, idx2d,
          ones_v):
        c = lax.axis_index("c")
        s = lax.axis_index("s")
        base_e = s * _EPT
        pltpu.sync_copy(ei_hbm.at[pl.ds(base_e, _EPT)], srcv)
        pltpu.sync_copy(ei_hbm.at[pl.ds(E + base_e, _EPT)], dstv)
        pltpu.sync_copy(ones_hbm, ones_v)

        for p in range(_PASSES):
            rb = p * (2 * _ROWS) + c * _ROWS

            pltpu.sync_copy(zeros_hbm.at[pl.ds(0, _WPT)],
                            acc.at[pl.ds(s * _WPT, _WPT)])
            pltpu.sync_copy(
                zeros_hbm.at[pl.ds(_WPT, _DUMMY // 16)],
                acc.at[pl.ds(_ROWS * N + s * (_DUMMY // 16), _DUMMY // 16)])
            plsc.subcore_barrier()

            def chunk(j, carry):
                for i in range(8):
                    off = j * 128 + i * 16
                    d = dstv[pl.ds(off, 16)]
                    sv = srcv[pl.ds(off, 16)]
                    rel = d - rb
                    mask = (rel >= 0) & (rel < _ROWS)
                    flat = rel * N + sv
                    spread = _ROWS * N + (
                        (off + lax.iota(jnp.int32, 16)) & (_DUMMY - 1))
                    idx2d[j, pl.ds(i * 16, 16)] = jnp.where(mask, flat, spread)
                pltpu.sync_copy(ones_v, acc.at[idx2d.at[j]], add=True)
                return carry
            lax.fori_loop(0, _EPT // 128, chunk, 0)
            plsc.subcore_barrier()

            pltpu.sync_copy(
                acc.at[pl.ds(s * _WPT, _WPT)],
                a0_hbm.at[pl.ds((rb + s * (_ROWS // 16)) * N, _WPT)])
            plsc.subcore_barrier()

    zeros = jnp.zeros((_WPT + _DUMMY // 16,), jnp.float32)
    ones = jnp.ones((128,), jnp.float32)
    return k(edge_index.reshape(-1), zeros, ones).reshape(N, N)


def kernel(edge_index_1, edge_index_2, feature, W1, b1, W2, b2, W3, b3,
           W4, b4, W5, b5, W6, b6):
    params = (W1, b1, W2, b2, W3, b3, W4, b4, W5, b5, W6, b6)
    a0_1 = _build_a0(edge_index_1)
    a0_2 = _build_a0(edge_index_2)
    t1 = _mm(feature, W1, bf16=True)
    fa, fabf = _conv_pass(t1, a0_1, params)
    fb, fbbf = _conv_pass(t1, a0_2, params)
    pred = _mm(fabf, fbbf, trans_lhs=True, bf16=True, bm=1024, bn=1024)
    return (fa, fb, pred)


# speedup vs baseline: 1.1758x; 1.1758x over previous
import functools

import jax
import jax.numpy as jnp
from jax import lax
from jax.experimental import pallas as pl
from jax.experimental.pallas import tpu as pltpu
from jax.experimental.pallas import tpu_sc as plsc

N = 4096
D = 512
E = 131072


def _mm_body(a_ref, b_ref, scale_ref, bias_ref, *refs, trans_lhs,
             scale_rows, bias, leaky, bm, bf16, out_bf16):
    if out_bf16:
        out_ref, out2_ref, acc_ref = refs
    else:
        out_ref, acc_ref = refs
    k = pl.program_id(2)

    @pl.when(k == 0)
    def _():
        acc_ref[...] = jnp.zeros_like(acc_ref)

    a, b = a_ref[...], b_ref[...]
    if bf16:
        a, b = a.astype(jnp.bfloat16), b.astype(jnp.bfloat16)
    if trans_lhs:
        acc_ref[...] += jax.lax.dot_general(
            a, b, (((0,), (0,)), ((), ())),
            preferred_element_type=jnp.float32)
    else:
        acc_ref[...] += jnp.dot(a, b, preferred_element_type=jnp.float32)

    @pl.when(k == pl.num_programs(2) - 1)
    def _():
        acc = acc_ref[...]
        if bias:
            j = pl.program_id(1)
            bn = out_ref.shape[1]
            acc = acc + bias_ref[pl.ds(j * bn, bn)][None, :]
        if scale_rows:
            i = pl.program_id(0)
            acc = acc * scale_ref[pl.ds(i * bm, bm)][:, None]
        if leaky:
            acc = jnp.where(acc > 0, acc, 0.01 * acc)
        out_ref[...] = acc
        if out_bf16:
            out2_ref[...] = acc.astype(jnp.bfloat16)


def _mm(a, b, *, scale=None, bias=None, leaky=False, trans_lhs=False,
        bf16=False, out_bf16=False, bm=512, bn=512, bk=512):
    if trans_lhs:
        ka, m = a.shape
    else:
        m, ka = a.shape
    kb, n = b.shape
    assert ka == kb
    grid = (m // bm, n // bn, ka // bk)
    in_specs = [
        pl.BlockSpec((bk, bm) if trans_lhs else (bm, bk),
                     (lambda i, j, k: (k, i)) if trans_lhs
                     else (lambda i, j, k: (i, k))),
        pl.BlockSpec((bk, bn), lambda i, j, k: (k, j)),
        pl.BlockSpec((m,), lambda i, j, k: (0,)),
        pl.BlockSpec((n,), lambda i, j, k: (0,)),
    ]
    scale_arr = scale if scale is not None else jnp.zeros((m,), jnp.float32)
    bias_arr = bias if bias is not None else jnp.zeros((n,), jnp.float32)
    body = functools.partial(_mm_body, trans_lhs=trans_lhs,
                             scale_rows=scale is not None,
                             bias=bias is not None, leaky=leaky, bm=bm,
                             bf16=bf16, out_bf16=out_bf16)
    out_specs = [pl.BlockSpec((bm, bn), lambda i, j, k: (i, j))]
    out_shape = [jax.ShapeDtypeStruct((m, n), jnp.float32)]
    if out_bf16:
        out_specs.append(pl.BlockSpec((bm, bn), lambda i, j, k: (i, j)))
        out_shape.append(jax.ShapeDtypeStruct((m, n), jnp.bfloat16))
    res = pl.pallas_call(
        body,
        grid=grid,
        in_specs=in_specs,
        out_specs=out_specs,
        out_shape=out_shape,
        scratch_shapes=[pltpu.VMEM((bm, bn), jnp.float32)],
        compiler_params=pltpu.CompilerParams(
            dimension_semantics=("parallel", "parallel", "arbitrary")),
    )(a, b, scale_arr, bias_arr)
    return res if out_bf16 else res[0]


def _agg_body(*refs, norm, leaky, bf16):
    if norm:
        a_ref, t_ref, td_ref, dis_ref, bias_ref, out_ref, acc_ref = refs
    else:
        a_ref, t_ref, out_ref, acc_ref = refs
    k = pl.program_id(0)

    @pl.when(k == 0)
    def _():
        acc_ref[...] = jnp.zeros_like(acc_ref)

    a, t = a_ref[...], t_ref[...]
    if bf16:
        a, t = a.astype(jnp.bfloat16), t.astype(jnp.bfloat16)
    acc_ref[...] += jnp.dot(a, t, preferred_element_type=jnp.float32)

    @pl.when(k == pl.num_programs(0) - 1)
    def _():
        acc = acc_ref[...]
        if norm:
            acc = (acc + td_ref[...]) * dis_ref[...][:, None]
            acc = acc + bias_ref[...][None, :]
        if leaky:
            acc = jnp.where(acc > 0, acc, 0.01 * acc)
        out_ref[...] = acc


def _agg(a0, ts, dis=None, bias=None, *, leaky=False, bf16=False, bk=512):
    n, d = ts.shape
    norm = dis is not None
    body = functools.partial(_agg_body, norm=norm, leaky=leaky, bf16=bf16)
    in_specs = [
        pl.BlockSpec((n, bk), lambda k: (0, k)),
        pl.BlockSpec((bk, d), lambda k: (k, 0)),
    ]
    args = [a0, ts]
    if norm:
        in_specs += [
            pl.BlockSpec((n, d), lambda k: (0, 0)),
            pl.BlockSpec((n,), lambda k: (0,)),
            pl.BlockSpec((d,), lambda k: (0,)),
        ]
        args += [ts, dis, bias]
    return pl.pallas_call(
        body,
        grid=(n // bk,),
        in_specs=in_specs,
        out_specs=pl.BlockSpec((n, d), lambda k: (0, 0)),
        out_shape=jax.ShapeDtypeStruct((n, d), jnp.float32),
        scratch_shapes=[pltpu.VMEM((n, d), jnp.float32)],
        compiler_params=pltpu.CompilerParams(
            dimension_semantics=("arbitrary",)),
    )(*args)


def _rowscale_body(t_ref, dis_ref, out_ref, *, bm):
    i = pl.program_id(0)
    out_ref[...] = t_ref[...] * dis_ref[pl.ds(i * bm, bm)][:, None]


def _rowscale(t, dis, *, bm=512):
    n, d = t.shape
    return pl.pallas_call(
        functools.partial(_rowscale_body, bm=bm),
        grid=(n // bm,),
        in_specs=[pl.BlockSpec((bm, d), lambda i: (i, 0)),
                  pl.BlockSpec((n,), lambda i: (0,))],
        out_specs=pl.BlockSpec((bm, d), lambda i: (i, 0)),
        out_shape=jax.ShapeDtypeStruct((n, d), jnp.float32),
    )(t, dis)


def _prep_body(a_ref, bf_ref, dis_ref):
    a = a_ref[...]
    bf_ref[...] = a.astype(jnp.bfloat16)
    dis_ref[...] = jax.lax.rsqrt(jnp.sum(a, axis=1) + 1.0)


def _prep(a0, *, bm=512):
    n = a0.shape[0]
    return pl.pallas_call(
        _prep_body,
        grid=(n // bm,),
        in_specs=[pl.BlockSpec((bm, n), lambda i: (i, 0))],
        out_specs=[pl.BlockSpec((bm, n), lambda i: (i, 0)),
                   pl.BlockSpec((bm,), lambda i: (i,))],
        out_shape=[jax.ShapeDtypeStruct((n, n), jnp.bfloat16),
                   jax.ShapeDtypeStruct((n,), jnp.float32)],
    )(a0)


def _conv_pass(t1, a0, params):
    (_, b1, W2, b2, W3, b3, W4, b4, W5, b5, W6, b6) = params
    a0bf, dis = _prep(a0)
    f = _agg(a0bf, _rowscale(t1, dis), dis, b1, leaky=False, bf16=True)
    for (W, b) in ((W2, b2), (W3, b3), (W4, b4), (W5, b5)):
        ts = _mm(f, W, scale=dis, bf16=True)
        f = _agg(a0bf, ts, dis, b, leaky=True, bf16=True)
    g = _agg(a0bf, f, bf16=True)
    return _mm(g, W6, bias=b6, leaky=True, bf16=True, bm=1024, bn=1024,
               out_bf16=True)


_ROWS = 256
_PASSES = N // (_ROWS * 2)
_EPT = E // 16
_WPT = _ROWS * N // 16
_DUMMY = 8192


def _build_a0(edge_index):
    mesh = plsc.VectorSubcoreMesh(core_axis_name="c", subcore_axis_name="s")

    @functools.partial(
        pl.kernel,
        out_type=jax.ShapeDtypeStruct((N, N), jnp.float32),
        mesh=mesh,
        scratch_types=[
            pltpu.VMEM_SHARED((_ROWS * N + _DUMMY,), jnp.float32),
            pltpu.VMEM((_EPT,), jnp.int32),
            pltpu.VMEM((_EPT,), jnp.int32),
            pltpu.VMEM((_EPT // 128, 128), jnp.int32),
            pltpu.VMEM((128,), jnp.float32),
            pltpu.SemaphoreType.DMA,
        ],
    )
    def k(ei_hbm, zeros_hbm, ones_hbm, a0_hbm, acc, srcv, dstv, idx2d,
          ones_v, sem):
        c = lax.axis_index("c")
        s = lax.axis_index("s")
        base_e = s * _EPT
        pltpu.sync_copy(ei_hbm.at[pl.ds(base_e, _EPT)], srcv)
        pltpu.sync_copy(ei_hbm.at[pl.ds(E + base_e, _EPT)], dstv)
        pltpu.sync_copy(ones_hbm, ones_v)

        for p in range(_PASSES):
            rb = p * (2 * _ROWS) + c * _ROWS

            pltpu.sync_copy(zeros_hbm.at[pl.ds(0, _WPT)],
                            acc.at[pl.ds(s * _WPT, _WPT)])
            pltpu.sync_copy(
                zeros_hbm.at[pl.ds(_WPT, _DUMMY // 16)],
                acc.at[pl.ds(_ROWS * N + s * (_DUMMY // 16), _DUMMY // 16)])
            plsc.subcore_barrier()

            def group(g, carry):
                handles = []
                for jj in range(8):
                    j = g * 8 + jj
                    for i in range(8):
                        off = j * 128 + i * 16
                        d = dstv[pl.ds(off, 16)]
                        sv = srcv[pl.ds(off, 16)]
                        rel = d - rb
                        mask = (rel >= 0) & (rel < _ROWS)
                        flat = rel * N + sv
                        spread = _ROWS * N + (
                            (off + lax.iota(jnp.int32, 16)) & (_DUMMY - 1))
                        idx2d[j, pl.ds(i * 16, 16)] = jnp.where(
                            mask, flat, spread)
                    handles.append(pltpu.async_copy(
                        ones_v, acc.at[idx2d.at[j]], sem, add=True))
                for h in handles:
                    h.wait()
                return carry
            lax.fori_loop(0, _EPT // 128 // 8, group, 0)
            plsc.subcore_barrier()

            r0 = rb + s * (_ROWS // 16)
            handles = []
            for r in range(_ROWS // 16):
                handles.append(pltpu.async_copy(
                    acc.at[pl.ds(s * _WPT + r * N, N)],
                    a0_hbm.at[r0 + r], sem))
            for h in handles:
                h.wait()
            plsc.subcore_barrier()

    zeros = jnp.zeros((_WPT + _DUMMY // 16,), jnp.float32)
    ones = jnp.ones((128,), jnp.float32)
    return k(edge_index.reshape(-1), zeros, ones)


def kernel(edge_index_1, edge_index_2, feature, W1, b1, W2, b2, W3, b3,
           W4, b4, W5, b5, W6, b6):
    params = (W1, b1, W2, b2, W3, b3, W4, b4, W5, b5, W6, b6)
    a0_1 = _build_a0(edge_index_1)
    a0_2 = _build_a0(edge_index_2)
    t1 = _mm(feature, W1, bf16=True)
    fa, fabf = _conv_pass(t1, a0_1, params)
    fb, fbbf = _conv_pass(t1, a0_2, params)
    pred = _mm(fabf, fbbf, trans_lhs=True, bf16=True, bm=1024, bn=1024)
    return (fa, fb, pred)


# speedup vs baseline: 1.2167x; 1.0347x over previous
import functools

import jax
import jax.numpy as jnp
from jax import lax
from jax.experimental import pallas as pl
from jax.experimental.pallas import tpu as pltpu
from jax.experimental.pallas import tpu_sc as plsc

N = 4096
D = 512
E = 131072


def _mm_body(a_ref, b_ref, scale_ref, bias_ref, *refs, trans_lhs,
             scale_rows, bias, leaky, bm, bf16, out_bf16):
    if out_bf16:
        out_ref, out2_ref, acc_ref = refs
    else:
        out_ref, acc_ref = refs
    k = pl.program_id(2)

    @pl.when(k == 0)
    def _():
        acc_ref[...] = jnp.zeros_like(acc_ref)

    a, b = a_ref[...], b_ref[...]
    if bf16:
        a, b = a.astype(jnp.bfloat16), b.astype(jnp.bfloat16)
    if trans_lhs:
        acc_ref[...] += jax.lax.dot_general(
            a, b, (((0,), (0,)), ((), ())),
            preferred_element_type=jnp.float32)
    else:
        acc_ref[...] += jnp.dot(a, b, preferred_element_type=jnp.float32)

    @pl.when(k == pl.num_programs(2) - 1)
    def _():
        acc = acc_ref[...]
        if bias:
            j = pl.program_id(1)
            bn = out_ref.shape[1]
            acc = acc + bias_ref[pl.ds(j * bn, bn)][None, :]
        if scale_rows:
            i = pl.program_id(0)
            acc = acc * scale_ref[pl.ds(i * bm, bm)][:, None]
        if leaky:
            acc = jnp.where(acc > 0, acc, 0.01 * acc)
        out_ref[...] = acc
        if out_bf16:
            out2_ref[...] = acc.astype(jnp.bfloat16)


def _mm(a, b, *, scale=None, bias=None, leaky=False, trans_lhs=False,
        bf16=False, out_bf16=False, bm=512, bn=512, bk=512):
    if trans_lhs:
        ka, m = a.shape
    else:
        m, ka = a.shape
    kb, n = b.shape
    assert ka == kb
    grid = (m // bm, n // bn, ka // bk)
    in_specs = [
        pl.BlockSpec((bk, bm) if trans_lhs else (bm, bk),
                     (lambda i, j, k: (k, i)) if trans_lhs
                     else (lambda i, j, k: (i, k))),
        pl.BlockSpec((bk, bn), lambda i, j, k: (k, j)),
        pl.BlockSpec((m,), lambda i, j, k: (0,)),
        pl.BlockSpec((n,), lambda i, j, k: (0,)),
    ]
    scale_arr = scale if scale is not None else jnp.zeros((m,), jnp.float32)
    bias_arr = bias if bias is not None else jnp.zeros((n,), jnp.float32)
    body = functools.partial(_mm_body, trans_lhs=trans_lhs,
                             scale_rows=scale is not None,
                             bias=bias is not None, leaky=leaky, bm=bm,
                             bf16=bf16, out_bf16=out_bf16)
    out_specs = [pl.BlockSpec((bm, bn), lambda i, j, k: (i, j))]
    out_shape = [jax.ShapeDtypeStruct((m, n), jnp.float32)]
    if out_bf16:
        out_specs.append(pl.BlockSpec((bm, bn), lambda i, j, k: (i, j)))
        out_shape.append(jax.ShapeDtypeStruct((m, n), jnp.bfloat16))
    res = pl.pallas_call(
        body,
        grid=grid,
        in_specs=in_specs,
        out_specs=out_specs,
        out_shape=out_shape,
        scratch_shapes=[pltpu.VMEM((bm, bn), jnp.float32)],
        compiler_params=pltpu.CompilerParams(
            dimension_semantics=("parallel", "parallel", "arbitrary")),
    )(a, b, scale_arr, bias_arr)
    return res if out_bf16 else res[0]


def _agg_body(*refs, norm, leaky, bf16):
    if norm:
        a_ref, t_ref, td_ref, dis_ref, bias_ref, out_ref, acc_ref = refs
    else:
        a_ref, t_ref, out_ref, acc_ref = refs
    k = pl.program_id(0)

    @pl.when(k == 0)
    def _():
        acc_ref[...] = jnp.zeros_like(acc_ref)

    a, t = a_ref[...], t_ref[...]
    if bf16:
        a, t = a.astype(jnp.bfloat16), t.astype(jnp.bfloat16)
    acc_ref[...] += jnp.dot(a, t, preferred_element_type=jnp.float32)

    @pl.when(k == pl.num_programs(0) - 1)
    def _():
        acc = acc_ref[...]
        if norm:
            acc = (acc + td_ref[...]) * dis_ref[...][:, None]
            acc = acc + bias_ref[...][None, :]
        if leaky:
            acc = jnp.where(acc > 0, acc, 0.01 * acc)
        out_ref[...] = acc


def _agg(a0, ts, dis=None, bias=None, *, leaky=False, bf16=False, bk=512):
    n, d = ts.shape
    norm = dis is not None
    body = functools.partial(_agg_body, norm=norm, leaky=leaky, bf16=bf16)
    in_specs = [
        pl.BlockSpec((n, bk), lambda k: (0, k)),
        pl.BlockSpec((bk, d), lambda k: (k, 0)),
    ]
    args = [a0, ts]
    if norm:
        in_specs += [
            pl.BlockSpec((n, d), lambda k: (0, 0)),
            pl.BlockSpec((n,), lambda k: (0,)),
            pl.BlockSpec((d,), lambda k: (0,)),
        ]
        args += [ts, dis, bias]
    return pl.pallas_call(
        body,
        grid=(n // bk,),
        in_specs=in_specs,
        out_specs=pl.BlockSpec((n, d), lambda k: (0, 0)),
        out_shape=jax.ShapeDtypeStruct((n, d), jnp.float32),
        scratch_shapes=[pltpu.VMEM((n, d), jnp.float32)],
        compiler_params=pltpu.CompilerParams(
            dimension_semantics=("arbitrary",)),
    )(*args)


def _rowscale_body(t_ref, dis_ref, out_ref, *, bm):
    i = pl.program_id(0)
    out_ref[...] = t_ref[...] * dis_ref[pl.ds(i * bm, bm)][:, None]


def _rowscale(t, dis, *, bm=512):
    n, d = t.shape
    return pl.pallas_call(
        functools.partial(_rowscale_body, bm=bm),
        grid=(n // bm,),
        in_specs=[pl.BlockSpec((bm, d), lambda i: (i, 0)),
                  pl.BlockSpec((n,), lambda i: (0,))],
        out_specs=pl.BlockSpec((bm, d), lambda i: (i, 0)),
        out_shape=jax.ShapeDtypeStruct((n, d), jnp.float32),
    )(t, dis)


def _prep_body(a_ref, bf_ref, dis_ref):
    a = a_ref[...]
    bf_ref[...] = a.astype(jnp.bfloat16)
    dis_ref[...] = jax.lax.rsqrt(jnp.sum(a, axis=1) + 1.0)


def _prep(a0, *, bm=512):
    n = a0.shape[0]
    return pl.pallas_call(
        _prep_body,
        grid=(n // bm,),
        in_specs=[pl.BlockSpec((bm, n), lambda i: (i, 0))],
        out_specs=[pl.BlockSpec((bm, n), lambda i: (i, 0)),
                   pl.BlockSpec((bm,), lambda i: (i,))],
        out_shape=[jax.ShapeDtypeStruct((n, n), jnp.bfloat16),
                   jax.ShapeDtypeStruct((n,), jnp.float32)],
    )(a0)


def _conv_pass(t1, a0, params):
    (_, b1, W2, b2, W3, b3, W4, b4, W5, b5, W6, b6) = params
    a0bf, dis = _prep(a0)
    f = _agg(a0bf, _rowscale(t1, dis), dis, b1, leaky=False, bf16=True)
    for (W, b) in ((W2, b2), (W3, b3), (W4, b4), (W5, b5)):
        ts = _mm(f, W, scale=dis, bf16=True)
        f = _agg(a0bf, ts, dis, b, leaky=True, bf16=True)
    g = _agg(a0bf, f, bf16=True)
    return _mm(g, W6, bias=b6, leaky=True, bf16=True, bm=1024, bn=1024,
               out_bf16=True)


_ROWS = 256
_PASSES = N // (_ROWS * 2)
_EPT = E // 16
_WPT = _ROWS * N // 16
_DUMMY = 8192


def _build_a0(edge_index):
    mesh = plsc.VectorSubcoreMesh(core_axis_name="c", subcore_axis_name="s")

    @functools.partial(
        pl.kernel,
        out_type=jax.ShapeDtypeStruct((N, N), jnp.float32),
        mesh=mesh,
        scratch_types=[
            pltpu.VMEM_SHARED((_ROWS * N + _DUMMY,), jnp.float32),
            pltpu.VMEM((_EPT,), jnp.int32),
            pltpu.VMEM((_EPT,), jnp.int32),
            pltpu.VMEM((_EPT // 128, 128), jnp.int32),
            pltpu.VMEM((128,), jnp.float32),
            pltpu.SemaphoreType.DMA,
        ],
    )
    def k(ei_hbm, zeros_hbm, ones_hbm, a0_hbm, acc, srcv, dstv, idx2d,
          ones_v, sem):
        c = lax.axis_index("c")
        s = lax.axis_index("s")
        base_e = s * _EPT
        pltpu.sync_copy(ei_hbm.at[pl.ds(base_e, _EPT)], srcv)
        pltpu.sync_copy(ei_hbm.at[pl.ds(E + base_e, _EPT)], dstv)
        pltpu.sync_copy(ones_hbm, ones_v)

        for p in range(_PASSES):
            rb = p * (2 * _ROWS) + c * _ROWS

            pltpu.sync_copy(zeros_hbm.at[pl.ds(0, _WPT)],
                            acc.at[pl.ds(s * _WPT, _WPT)])
            pltpu.sync_copy(
                zeros_hbm.at[pl.ds(_WPT, _DUMMY // 16)],
                acc.at[pl.ds(_ROWS * N + s * (_DUMMY // 16), _DUMMY // 16)])
            plsc.subcore_barrier()

            def group(g, carry):
                handles = []
                for jj in range(8):
                    j = g * 8 + jj
                    for i in range(8):
                        off = j * 128 + i * 16
                        d = dstv[pl.ds(off, 16)]
                        sv = srcv[pl.ds(off, 16)]
                        rel = d - rb
                        mask = (rel >= 0) & (rel < _ROWS)
                        flat = rel * N + sv
                        spread = _ROWS * N + (
                            (off + lax.iota(jnp.int32, 16)) & (_DUMMY - 1))
                        idx2d[j, pl.ds(i * 16, 16)] = jnp.where(
                            mask, flat, spread)
                    handles.append(pltpu.async_copy(
                        ones_v, acc.at[idx2d.at[j]], sem, add=True))
                for h in handles:
                    h.wait()
                return carry
            lax.fori_loop(0, _EPT // 128 // 8, group, 0)
            plsc.subcore_barrier()

            r0 = rb + s * (_ROWS // 16)
            handles = []
            for r in range(_ROWS // 16):
                handles.append(pltpu.async_copy(
                    acc.at[pl.ds(s * _WPT + r * N, N)],
                    a0_hbm.at[r0 + r], sem))
            for h in handles:
                h.wait()
            plsc.subcore_barrier()

    zeros = jnp.zeros((_WPT + _DUMMY // 16,), jnp.float32)
    ones = jnp.ones((128,), jnp.float32)
    return k(edge_index.reshape(-1), zeros, ones)


def kernel(edge_index_1, edge_index_2, feature, W1, b1, W2, b2, W3, b3,
           W4, b4, W5, b5, W6, b6):
    params = (W1, b1, W2, b2, W3, b3, W4, b4, W5, b5, W6, b6)
    a0_1 = _build_a0(edge_index_1)
    a0_2 = _build_a0(edge_index_2)
    t1 = _mm(feature, W1, bf16=True, bm=1024, bk=1024)
    fa, fabf = _conv_pass(t1, a0_1, params)
    fb, fbbf = _conv_pass(t1, a0_2, params)
    pred = _mm(fabf, fbbf, trans_lhs=True, bf16=True, bm=2048, bn=1024)
    return (fa, fb, pred)


# speedup vs baseline: 1.2415x; 1.0204x over previous
import functools

import jax
import jax.numpy as jnp
from jax import lax
from jax.experimental import pallas as pl
from jax.experimental.pallas import tpu as pltpu
from jax.experimental.pallas import tpu_sc as plsc

N = 4096
D = 512
E = 131072


def _mm_body(a_ref, b_ref, scale_ref, bias_ref, *refs, trans_lhs,
             scale_rows, bias, leaky, bm, bf16, out_bf16):
    if out_bf16:
        out_ref, out2_ref, acc_ref = refs
    else:
        out_ref, acc_ref = refs
    k = pl.program_id(2)

    @pl.when(k == 0)
    def _():
        acc_ref[...] = jnp.zeros_like(acc_ref)

    a, b = a_ref[...], b_ref[...]
    if bf16:
        a, b = a.astype(jnp.bfloat16), b.astype(jnp.bfloat16)
    if trans_lhs:
        acc_ref[...] += jax.lax.dot_general(
            a, b, (((0,), (0,)), ((), ())),
            preferred_element_type=jnp.float32)
    else:
        acc_ref[...] += jnp.dot(a, b, preferred_element_type=jnp.float32)

    @pl.when(k == pl.num_programs(2) - 1)
    def _():
        acc = acc_ref[...]
        if bias:
            j = pl.program_id(1)
            bn = out_ref.shape[1]
            acc = acc + bias_ref[pl.ds(j * bn, bn)][None, :]
        if scale_rows:
            i = pl.program_id(0)
            acc = acc * scale_ref[pl.ds(i * bm, bm)][:, None]
        if leaky:
            acc = jnp.where(acc > 0, acc, 0.01 * acc)
        out_ref[...] = acc
        if out_bf16:
            out2_ref[...] = acc.astype(jnp.bfloat16)


def _mm(a, b, *, scale=None, bias=None, leaky=False, trans_lhs=False,
        bf16=False, out_bf16=False, bm=512, bn=512, bk=512):
    if trans_lhs:
        ka, m = a.shape
    else:
        m, ka = a.shape
    kb, n = b.shape
    assert ka == kb
    grid = (m // bm, n // bn, ka // bk)
    in_specs = [
        pl.BlockSpec((bk, bm) if trans_lhs else (bm, bk),
                     (lambda i, j, k: (k, i)) if trans_lhs
                     else (lambda i, j, k: (i, k))),
        pl.BlockSpec((bk, bn), lambda i, j, k: (k, j)),
        pl.BlockSpec((m,), lambda i, j, k: (0,)),
        pl.BlockSpec((n,), lambda i, j, k: (0,)),
    ]
    scale_arr = scale if scale is not None else jnp.zeros((m,), jnp.float32)
    bias_arr = bias if bias is not None else jnp.zeros((n,), jnp.float32)
    body = functools.partial(_mm_body, trans_lhs=trans_lhs,
                             scale_rows=scale is not None,
                             bias=bias is not None, leaky=leaky, bm=bm,
                             bf16=bf16, out_bf16=out_bf16)
    out_specs = [pl.BlockSpec((bm, bn), lambda i, j, k: (i, j))]
    out_shape = [jax.ShapeDtypeStruct((m, n), jnp.float32)]
    if out_bf16:
        out_specs.append(pl.BlockSpec((bm, bn), lambda i, j, k: (i, j)))
        out_shape.append(jax.ShapeDtypeStruct((m, n), jnp.bfloat16))
    res = pl.pallas_call(
        body,
        grid=grid,
        in_specs=in_specs,
        out_specs=out_specs,
        out_shape=out_shape,
        scratch_shapes=[pltpu.VMEM((bm, bn), jnp.float32)],
        compiler_params=pltpu.CompilerParams(
            dimension_semantics=("parallel", "parallel", "arbitrary")),
    )(a, b, scale_arr, bias_arr)
    return res if out_bf16 else res[0]


def _agg_body(*refs, norm, leaky, bf16):
    if norm:
        a_ref, t_ref, td_ref, dis_ref, bias_ref, out_ref, acc_ref = refs
    else:
        a_ref, t_ref, out_ref, acc_ref = refs
    k = pl.program_id(0)

    @pl.when(k == 0)
    def _():
        acc_ref[...] = jnp.zeros_like(acc_ref)

    a, t = a_ref[...], t_ref[...]
    if bf16:
        a, t = a.astype(jnp.bfloat16), t.astype(jnp.bfloat16)
    acc_ref[...] += jnp.dot(a, t, preferred_element_type=jnp.float32)

    @pl.when(k == pl.num_programs(0) - 1)
    def _():
        acc = acc_ref[...]
        if norm:
            acc = (acc + td_ref[...]) * dis_ref[...][:, None]
            acc = acc + bias_ref[...][None, :]
        if leaky:
            acc = jnp.where(acc > 0, acc, 0.01 * acc)
        out_ref[...] = acc


def _agg(a0, ts, dis=None, bias=None, *, leaky=False, bf16=False, bk=512):
    n, d = ts.shape
    norm = dis is not None
    body = functools.partial(_agg_body, norm=norm, leaky=leaky, bf16=bf16)
    in_specs = [
        pl.BlockSpec((n, bk), lambda k: (0, k)),
        pl.BlockSpec((bk, d), lambda k: (k, 0)),
    ]
    args = [a0, ts]
    if norm:
        in_specs += [
            pl.BlockSpec((n, d), lambda k: (0, 0)),
            pl.BlockSpec((n,), lambda k: (0,)),
            pl.BlockSpec((d,), lambda k: (0,)),
        ]
        args += [ts, dis, bias]
    return pl.pallas_call(
        body,
        grid=(n // bk,),
        in_specs=in_specs,
        out_specs=pl.BlockSpec((n, d), lambda k: (0, 0)),
        out_shape=jax.ShapeDtypeStruct((n, d), jnp.float32),
        scratch_shapes=[pltpu.VMEM((n, d), jnp.float32)],
        compiler_params=pltpu.CompilerParams(
            dimension_semantics=("arbitrary",)),
    )(*args)


def _rowscale_body(t_ref, dis_ref, out_ref, *, bm):
    i = pl.program_id(0)
    out_ref[...] = t_ref[...] * dis_ref[pl.ds(i * bm, bm)][:, None]


def _rowscale(t, dis, *, bm=512):
    n, d = t.shape
    return pl.pallas_call(
        functools.partial(_rowscale_body, bm=bm),
        grid=(n // bm,),
        in_specs=[pl.BlockSpec((bm, d), lambda i: (i, 0)),
                  pl.BlockSpec((n,), lambda i: (0,))],
        out_specs=pl.BlockSpec((bm, d), lambda i: (i, 0)),
        out_shape=jax.ShapeDtypeStruct((n, d), jnp.float32),
    )(t, dis)


def _prep_body(a_ref, bf_ref, dis_ref):
    a = a_ref[...]
    bf_ref[...] = a.astype(jnp.bfloat16)
    dis_ref[...] = jax.lax.rsqrt(jnp.sum(a, axis=1) + 1.0)


def _prep(a0, *, bm=512):
    n = a0.shape[0]
    return pl.pallas_call(
        _prep_body,
        grid=(n // bm,),
        in_specs=[pl.BlockSpec((bm, n), lambda i: (i, 0))],
        out_specs=[pl.BlockSpec((bm, n), lambda i: (i, 0)),
                   pl.BlockSpec((bm,), lambda i: (i,))],
        out_shape=[jax.ShapeDtypeStruct((n, n), jnp.bfloat16),
                   jax.ShapeDtypeStruct((n,), jnp.float32)],
    )(a0)


def _conv_pass(t1, a0, params):
    (_, b1, W2, b2, W3, b3, W4, b4, W5, b5, W6, b6) = params
    a0bf, dis = _prep(a0)
    f = _agg(a0bf, _rowscale(t1, dis), dis, b1, leaky=False, bf16=True)
    for (W, b) in ((W2, b2), (W3, b3), (W4, b4), (W5, b5)):
        ts = _mm(f, W, scale=dis, bf16=True)
        f = _agg(a0bf, ts, dis, b, leaky=True, bf16=True)
    g = _agg(a0bf, f, bf16=True)
    return _mm(g, W6, bias=b6, leaky=True, bf16=True, bm=1024, bn=1024,
               out_bf16=True)


_ROWS = 384
_ROWS_LAST = 128
_EPT = E // 16
_DUMMY = 8192


def _build_a0(edge_index):
    mesh = plsc.VectorSubcoreMesh(core_axis_name="c", subcore_axis_name="s")

    @functools.partial(
        pl.kernel,
        out_type=jax.ShapeDtypeStruct((N, N), jnp.float32),
        mesh=mesh,
        scratch_types=[
            pltpu.VMEM_SHARED((_ROWS * N + _DUMMY,), jnp.float32),
            pltpu.VMEM((_EPT,), jnp.int32),
            pltpu.VMEM((_EPT,), jnp.int32),
            pltpu.VMEM((_EPT // 128, 128), jnp.int32),
            pltpu.VMEM((128,), jnp.float32),
            pltpu.SemaphoreType.DMA,
        ],
    )
    def k(ei_hbm, zeros_hbm, ones_hbm, a0_hbm, acc, srcv, dstv, idx2d,
          ones_v, sem):
        c = lax.axis_index("c")
        s = lax.axis_index("s")
        base_e = s * _EPT
        pltpu.sync_copy(ei_hbm.at[pl.ds(base_e, _EPT)], srcv)
        pltpu.sync_copy(ei_hbm.at[pl.ds(E + base_e, _EPT)], dstv)
        pltpu.sync_copy(ones_hbm, ones_v)

        for p in range(6):
            rows_p = _ROWS if p < 5 else _ROWS_LAST
            wpt = rows_p * N // 16
            rb = p * (2 * _ROWS) + c * rows_p

            pltpu.sync_copy(zeros_hbm.at[pl.ds(0, wpt)],
                            acc.at[pl.ds(s * wpt, wpt)])
            pltpu.sync_copy(
                zeros_hbm.at[pl.ds(wpt, _DUMMY // 16)],
                acc.at[pl.ds(_ROWS * N + s * (_DUMMY // 16), _DUMMY // 16)])
            plsc.subcore_barrier()

            def group(g, carry):
                handles = []
                for jj in range(8):
                    j = g * 8 + jj
                    for i in range(8):
                        off = j * 128 + i * 16
                        d = dstv[pl.ds(off, 16)]
                        sv = srcv[pl.ds(off, 16)]
                        rel = d - rb
                        mask = (rel >= 0) & (rel < rows_p)
                        flat = rel * N + sv
                        spread = _ROWS * N + (
                            (off + lax.iota(jnp.int32, 16)) & (_DUMMY - 1))
                        idx2d[j, pl.ds(i * 16, 16)] = jnp.where(
                            mask, flat, spread)
                    handles.append(pltpu.async_copy(
                        ones_v, acc.at[idx2d.at[j]], sem, add=True))
                for h in handles:
                    h.wait()
                return carry
            lax.fori_loop(0, _EPT // 128 // 8, group, 0)
            plsc.subcore_barrier()

            r0 = rb + s * (rows_p // 16)
            handles = []
            for r in range(rows_p // 16):
                handles.append(pltpu.async_copy(
                    acc.at[pl.ds(s * wpt + r * N, N)],
                    a0_hbm.at[r0 + r], sem))
            for h in handles:
                h.wait()
            plsc.subcore_barrier()

    zeros = jnp.zeros((_ROWS * N // 16 + _DUMMY // 16,), jnp.float32)
    ones = jnp.ones((128,), jnp.float32)
    return k(edge_index.reshape(-1), zeros, ones)


def kernel(edge_index_1, edge_index_2, feature, W1, b1, W2, b2, W3, b3,
           W4, b4, W5, b5, W6, b6):
    params = (W1, b1, W2, b2, W3, b3, W4, b4, W5, b5, W6, b6)
    a0_1 = _build_a0(edge_index_1)
    a0_2 = _build_a0(edge_index_2)
    t1 = _mm(feature, W1, bf16=True, bm=1024, bk=1024)
    fa, fabf = _conv_pass(t1, a0_1, params)
    fb, fbbf = _conv_pass(t1, a0_2, params)
    pred = _mm(fabf, fbbf, trans_lhs=True, bf16=True, bm=2048, bn=1024)
    return (fa, fb, pred)
